# Initial kernel scaffold; baseline (speedup 1.0000x reference)
import jax
import jax.numpy as jnp
from jax.experimental import pallas as pl


def kernel(x, router_w, router_b, expert_biases, sw1, sb1, sw2, sb2, sw3, sb3, rw1, rb1, rw2, rb2, rw3, rb3):
    raise NotImplementedError("write your pallas kernel here")


# speedup vs baseline: 2.2491x; 2.2491x over previous
import functools

import jax
import jax.numpy as jnp
from jax import lax
from jax.experimental import pallas as pl
from jax.experimental.pallas import tpu as pltpu
from jax.experimental.pallas import tpu_sc as plsc

S, H, FF, E = 2048, 1024, 2816, 8
T = 256
NT = 24
NPAD = NT * T
NP = 2 * S
NFC = 2
FFC = FF // NFC
ST = S // T


def _vgather16(v, idx):
    dn = lax.GatherDimensionNumbers(offset_dims=(), collapsed_slice_dims=(0,),
                                    start_index_map=(0,))
    return lax.gather(v, idx[:, None], dn, slice_sizes=(1,),
                      mode=lax.GatherScatterMode.PROMISE_IN_BOUNDS)


def _router_body(x_ref, rw_ref, rb_ref, eb_ref, i1_ref, i2_ref, g1_ref, g2_ref):
    x = x_ref[...]
    logits = lax.dot_general(x, rw_ref[...], (((1,), (1,)), ((), ())),
                             preferred_element_type=jnp.float32)
    logits = logits + rb_ref[...][None, :]
    lb = logits + eb_ref[...][None, :]
    eio = lax.broadcasted_iota(jnp.int32, lb.shape, 1)
    big = jnp.int32(1 << 30)
    m1 = jnp.max(lb, axis=-1, keepdims=True)
    i1 = jnp.min(jnp.where(lb == m1, eio, big), axis=-1, keepdims=True)
    lb2 = jnp.where(eio == i1, -jnp.inf, lb)
    m2 = jnp.max(lb2, axis=-1, keepdims=True)
    i2 = jnp.min(jnp.where(lb2 == m2, eio, big), axis=-1, keepdims=True)
    s1 = jnp.sum(jnp.where(eio == i1, logits, 0.0), axis=-1)
    s2 = jnp.sum(jnp.where(eio == i2, logits, 0.0), axis=-1)
    p1 = jax.nn.sigmoid(s1)
    p2 = jax.nn.sigmoid(s2)
    den = jnp.maximum(p1 + p2, 1e-12)
    i1_ref[...] = i1[:, 0]
    i2_ref[...] = i2[:, 0]
    g1_ref[...] = p1 / den
    g2_ref[...] = p2 / den


def _router(x2d, router_w, router_b, expert_biases):
    return pl.pallas_call(
        _router_body,
        out_shape=[
            jax.ShapeDtypeStruct((S,), jnp.int32),
            jax.ShapeDtypeStruct((S,), jnp.int32),
            jax.ShapeDtypeStruct((S,), jnp.float32),
            jax.ShapeDtypeStruct((S,), jnp.float32),
        ],
    )(x2d, router_w, router_b, expert_biases)


def _dispatch(i1, i2, g1, g2, x2d):
    mesh = plsc.VectorSubcoreMesh(core_axis_name="c", subcore_axis_name="s")
    out_type = [
        jax.ShapeDtypeStruct((NPAD, H), jnp.float32),
        jax.ShapeDtypeStruct((NPAD,), jnp.float32),
        jax.ShapeDtypeStruct((32,), jnp.int32),
        jax.ShapeDtypeStruct((S,), jnp.int32),
        jax.ShapeDtypeStruct((S,), jnp.int32),
    ]
    scratch = [
        pltpu.VMEM((NP,), jnp.int32),
        pltpu.VMEM((NP,), jnp.float32),
        pltpu.VMEM((NP,), jnp.int32),
        pltpu.VMEM((NP,), jnp.int32),
        pltpu.VMEM((NPAD,), jnp.int32),
        pltpu.VMEM((NPAD,), jnp.float32),
        pltpu.VMEM((32,), jnp.int32),
        pltpu.VMEM_SHARED((NPAD,), jnp.int32),
        pltpu.VMEM((64,), jnp.int32),
        pltpu.VMEM((64, H), jnp.float32),
        pltpu.SemaphoreType.DMA,
    ]

    @functools.partial(
        pl.kernel, mesh=mesh, out_type=out_type, scratch_types=scratch,
        compiler_params=pltpu.CompilerParams(needs_layout_passes=False))
    def body(i1h, i2h, g1h, g2h, xh, xs, gs, te, pos1, pos2,
             ep, gp, ranks, posb, ptok, gsort, tebuf, shp,
             myidx, rows, sem):
        c = lax.axis_index("c")
        s = lax.axis_index("s")
        io16 = lax.iota(jnp.int32, 16)

        @pl.when(s == 0)
        def _leader():
            pltpu.sync_copy(i1h, ep.at[pl.ds(0, S)])
            pltpu.sync_copy(i2h, ep.at[pl.ds(S, S)])
            pltpu.sync_copy(g1h, gp.at[pl.ds(0, S)])
            pltpu.sync_copy(g2h, gp.at[pl.ds(S, S)])
            zi = jnp.zeros((16,), jnp.int32)
            zf = jnp.zeros((16,), jnp.float32)

            def init_body(i, carry):
                ptok[pl.ds(i * 16, 16)] = zi
                gsort[pl.ds(i * 16, 16)] = zf
                return carry

            lax.fori_loop(0, NPAD // 16, init_body, 0)

            def group_a(g, carry):
                pidx, cnt = carry
                ev = ep[pl.ds(g * 16, 16)]
                cbase = _vgather16(cnt, ev)
                rank = jnp.zeros((16,), jnp.int32)
                tot = jnp.zeros((16,), jnp.int32)
                for e in range(E):
                    m = ev == e
                    mi = m.astype(jnp.int32)
                    incl = plsc.cumsum(mi)
                    rank = jnp.where(m, incl - 1, rank)
                    tote = plsc.cummax(lax.rev(incl, (0,)))
                    tot = jnp.where(io16 == e, tote, tot)
                ranks[pl.ds(g * 16, 16)] = rank + cbase
                return (pidx + 16, cnt + tot)

            _, counts_v = lax.fori_loop(
                0, NP // 16, group_a, (io16, jnp.zeros((16,), jnp.int32)))

            ntiles_v = (counts_v + (T - 1)) // T
            tstart_v = plsc.cumsum(ntiles_v) - ntiles_v
            basevals = tstart_v * T
            for j in range(2):
                tv = io16 + j * 16
                ind = jnp.zeros((16,), jnp.int32)
                for e in range(E):
                    tse = _vgather16(tstart_v, jnp.full((16,), e, jnp.int32))
                    ind = ind + jnp.where(tv >= tse, 1, 0)
                tebuf[pl.ds(j * 16, 16)] = jnp.maximum(ind - 1, 0)

            def group_b(g, pidx):
                ev = ep[pl.ds(g * 16, 16)]
                rk = ranks[pl.ds(g * 16, 16)]
                gv = gp[pl.ds(g * 16, 16)]
                base = _vgather16(basevals, ev)
                posv = base + rk
                tokv = pidx & (S - 1)
                plsc.store_scatter(ptok, [posv], tokv)
                plsc.store_scatter(gsort, [posv], gv)
                posb[pl.ds(g * 16, 16)] = posv
                return pidx + 16

            lax.fori_loop(0, NP // 16, group_b, io16)
            pltpu.sync_copy(ptok, shp)

            @pl.when(c == 0)
            def _meta():
                pltpu.sync_copy(posb.at[pl.ds(0, S)], pos1)
                pltpu.sync_copy(posb.at[pl.ds(S, S)], pos2)
                pltpu.sync_copy(gsort, gs)
                pltpu.sync_copy(tebuf, te)

        plsc.subcore_barrier()

        half = NPAD // 2
        per = half // 16
        for ch in range(per // 64):
            start = c * half + s * per + ch * 64
            pltpu.sync_copy(shp.at[pl.ds(start, 64)], myidx)
            pltpu.async_copy(xh.at[myidx], rows, sem).wait()
            pltpu.sync_copy(rows, xs.at[pl.ds(start, 64)])

    return body(i1, i2, g1, g2, x2d)


def _ffn_routed_body(te_ref, xs_ref, g_ref, w1_ref, b1_ref, w2_ref, b2_ref,
                     w3_ref, b3_ref, out_ref):
    j = pl.program_id(1)
    xt = xs_ref[...]
    gcol = g_ref[0, 0][:, None]
    h1 = lax.dot_general(xt, w1_ref[0], (((1,), (1,)), ((), ())),
                         preferred_element_type=jnp.float32) + b1_ref[0, 0]
    h3 = lax.dot_general(xt, w3_ref[0], (((1,), (1,)), ((), ())),
                         preferred_element_type=jnp.float32) + b3_ref[0, 0]
    hh = h1 * jax.nn.sigmoid(h1) * h3
    y = lax.dot_general(hh, w2_ref[0], (((1,), (1,)), ((), ())),
                        preferred_element_type=jnp.float32)

    @pl.when(j == 0)
    def _():
        out_ref[...] = (y + b2_ref[0]) * gcol

    @pl.when(j != 0)
    def _():
        out_ref[...] = out_ref[...] + y * gcol


def _ffn_routed(te, xs, gs3, rw1, rb1, rw2, rb2, rw3, rb3):
    grid_spec = pltpu.PrefetchScalarGridSpec(
        num_scalar_prefetch=1,
        grid=(NT, NFC),
        in_specs=[
            pl.BlockSpec((T, H), lambda i, j, te: (i, 0)),
            pl.BlockSpec((1, 1, T), lambda i, j, te: (i, 0, 0)),
            pl.BlockSpec((1, FFC, H), lambda i, j, te: (te[i], j, 0)),
            pl.BlockSpec((1, 1, 1, FFC), lambda i, j, te: (te[i], j, 0, 0)),
            pl.BlockSpec((1, H, FFC), lambda i, j, te: (te[i], 0, j)),
            pl.BlockSpec((1, 1, H), lambda i, j, te: (te[i], 0, 0)),
            pl.BlockSpec((1, FFC, H), lambda i, j, te: (te[i], j, 0)),
            pl.BlockSpec((1, 1, 1, FFC), lambda i, j, te: (te[i], j, 0, 0)),
        ],
        out_specs=pl.BlockSpec((T, H), lambda i, j, te: (i, 0)),
    )
    return pl.pallas_call(
        _ffn_routed_body,
        grid_spec=grid_spec,
        out_shape=jax.ShapeDtypeStruct((NPAD, H), jnp.float32),
        compiler_params=pltpu.CompilerParams(
            dimension_semantics=("arbitrary", "arbitrary"),
            vmem_limit_bytes=128 * 1024 * 1024,
        ),
    )(te, xs, gs3, rw1, rb1, rw2, rb2, rw3, rb3)


def _combine_gather(pos1, pos2, yr):
    mesh = plsc.VectorSubcoreMesh(core_axis_name="c", subcore_axis_name="s")
    out_type = [
        jax.ShapeDtypeStruct((S, H), jnp.float32),
        jax.ShapeDtypeStruct((S, H), jnp.float32),
    ]
    scratch = [
        pltpu.VMEM((64,), jnp.int32),
        pltpu.VMEM((64, H), jnp.float32),
        pltpu.SemaphoreType.DMA,
    ]

    @functools.partial(pl.kernel, mesh=mesh, out_type=out_type,
                       scratch_types=scratch)
    def body(p1h, p2h, yh, a, b, myidx, rows, sem):
        c = lax.axis_index("c")
        s = lax.axis_index("s")
        base = (c * 16 + s) * 64
        pltpu.sync_copy(p1h.at[pl.ds(base, 64)], myidx)
        pltpu.async_copy(yh.at[myidx], rows, sem).wait()
        pltpu.sync_copy(rows, a.at[pl.ds(base, 64)])
        pltpu.sync_copy(p2h.at[pl.ds(base, 64)], myidx)
        pltpu.async_copy(yh.at[myidx], rows, sem).wait()
        pltpu.sync_copy(rows, b.at[pl.ds(base, 64)])

    return body(pos1, pos2, yr)


def _ffn_shared_body(x_ref, w1_ref, b1_ref, w2_ref, b2_ref, w3_ref, b3_ref,
                     a_ref, b2r_ref, out_ref):
    j = pl.program_id(1)
    xt = x_ref[...]
    h1 = lax.dot_general(xt, w1_ref[...], (((1,), (1,)), ((), ())),
                         preferred_element_type=jnp.float32) + b1_ref[0]
    h3 = lax.dot_general(xt, w3_ref[...], (((1,), (1,)), ((), ())),
                         preferred_element_type=jnp.float32) + b3_ref[0]
    hh = h1 * jax.nn.sigmoid(h1) * h3
    y = lax.dot_general(hh, w2_ref[...], (((1,), (1,)), ((), ())),
                        preferred_element_type=jnp.float32)

    @pl.when(j == 0)
    def _():
        out_ref[...] = y + b2_ref[...][None, :] + a_ref[...] + b2r_ref[...]

    @pl.when(j != 0)
    def _():
        out_ref[...] = out_ref[...] + y


def _ffn_shared(x2d, sw1, sb1, sw2, sb2, sw3, sb3, a, b):
    return pl.pallas_call(
        _ffn_shared_body,
        grid=(ST, NFC),
        in_specs=[
            pl.BlockSpec((T, H), lambda i, j: (i, 0)),
            pl.BlockSpec((FFC, H), lambda i, j: (j, 0)),
            pl.BlockSpec((1, 1, FFC), lambda i, j: (j, 0, 0)),
            pl.BlockSpec((H, FFC), lambda i, j: (0, j)),
            pl.BlockSpec((H,), lambda i, j: (0,)),
            pl.BlockSpec((FFC, H), lambda i, j: (j, 0)),
            pl.BlockSpec((1, 1, FFC), lambda i, j: (j, 0, 0)),
            pl.BlockSpec((T, H), lambda i, j: (i, 0)),
            pl.BlockSpec((T, H), lambda i, j: (i, 0)),
        ],
        out_specs=pl.BlockSpec((T, H), lambda i, j: (i, 0)),
        out_shape=jax.ShapeDtypeStruct((S, H), jnp.float32),
        compiler_params=pltpu.CompilerParams(
            dimension_semantics=("arbitrary", "arbitrary"),
            vmem_limit_bytes=128 * 1024 * 1024,
        ),
    )(x2d, sw1, sb1, sw2, sb2, sw3, sb3, a, b)


def kernel(x, router_w, router_b, expert_biases, sw1, sb1, sw2, sb2, sw3, sb3,
           rw1, rb1, rw2, rb2, rw3, rb3):
    x2d = x.reshape(S, H)
    i1, i2, g1, g2 = _router(x2d, router_w, router_b, expert_biases)
    xs, gs, te, pos1, pos2 = _dispatch(i1, i2, g1, g2, x2d)
    gs3 = gs.reshape(NT, 1, T)
    rb1r = rb1.reshape(E, NFC, 1, FFC)
    rb3r = rb3.reshape(E, NFC, 1, FFC)
    rb2r = rb2.reshape(E, 1, H)
    yr = _ffn_routed(te, xs, gs3, rw1, rb1r, rw2, rb2r, rw3, rb3r)
    a, b = _combine_gather(pos1, pos2, yr)
    sb1r = sb1.reshape(NFC, 1, FFC)
    sb3r = sb3.reshape(NFC, 1, FFC)
    out = _ffn_shared(x2d, sw1, sb1r, sw2, sb2, sw3, sb3r, a, b)
    return out.reshape(1, S, H)


# speedup vs baseline: 2.2512x; 1.0009x over previous
import functools

import jax
import jax.numpy as jnp
from jax import lax
from jax.experimental import pallas as pl
from jax.experimental.pallas import tpu as pltpu
from jax.experimental.pallas import tpu_sc as plsc

S, H, FF, E = 2048, 1024, 2816, 8
T = 256
NT = 24
NPAD = NT * T
NP = 2 * S
NFC = 2
FFC = FF // NFC
ST = S // T


def _vgather16(v, idx):
    dn = lax.GatherDimensionNumbers(offset_dims=(), collapsed_slice_dims=(0,),
                                    start_index_map=(0,))
    return lax.gather(v, idx[:, None], dn, slice_sizes=(1,),
                      mode=lax.GatherScatterMode.PROMISE_IN_BOUNDS)


def _router_body(x_ref, rw_ref, rb_ref, eb_ref, i1_ref, i2_ref, g1_ref, g2_ref):
    x = x_ref[...]
    logits = lax.dot_general(x, rw_ref[...], (((1,), (1,)), ((), ())),
                             preferred_element_type=jnp.float32)
    logits = logits + rb_ref[...][None, :]
    lb = logits + eb_ref[...][None, :]
    eio = lax.broadcasted_iota(jnp.int32, lb.shape, 1)
    big = jnp.int32(1 << 30)
    m1 = jnp.max(lb, axis=-1, keepdims=True)
    i1 = jnp.min(jnp.where(lb == m1, eio, big), axis=-1, keepdims=True)
    lb2 = jnp.where(eio == i1, -jnp.inf, lb)
    m2 = jnp.max(lb2, axis=-1, keepdims=True)
    i2 = jnp.min(jnp.where(lb2 == m2, eio, big), axis=-1, keepdims=True)
    s1 = jnp.sum(jnp.where(eio == i1, logits, 0.0), axis=-1)
    s2 = jnp.sum(jnp.where(eio == i2, logits, 0.0), axis=-1)
    p1 = jax.nn.sigmoid(s1)
    p2 = jax.nn.sigmoid(s2)
    den = jnp.maximum(p1 + p2, 1e-12)
    i1_ref[...] = i1[:, 0]
    i2_ref[...] = i2[:, 0]
    g1_ref[...] = p1 / den
    g2_ref[...] = p2 / den


def _router(x2d, router_w, router_b, expert_biases):
    return pl.pallas_call(
        _router_body,
        out_shape=[
            jax.ShapeDtypeStruct((S,), jnp.int32),
            jax.ShapeDtypeStruct((S,), jnp.int32),
            jax.ShapeDtypeStruct((S,), jnp.float32),
            jax.ShapeDtypeStruct((S,), jnp.float32),
        ],
    )(x2d, router_w, router_b, expert_biases)


def _dispatch(i1, i2, g1, g2, x2d):
    mesh = plsc.VectorSubcoreMesh(core_axis_name="c", subcore_axis_name="s")
    out_type = [
        jax.ShapeDtypeStruct((NPAD, H), jnp.float32),
        jax.ShapeDtypeStruct((NPAD,), jnp.float32),
        jax.ShapeDtypeStruct((32,), jnp.int32),
        jax.ShapeDtypeStruct((S,), jnp.int32),
        jax.ShapeDtypeStruct((S,), jnp.int32),
    ]
    scratch = [
        pltpu.VMEM((NP,), jnp.int32),
        pltpu.VMEM((NP,), jnp.float32),
        pltpu.VMEM((NP,), jnp.int32),
        pltpu.VMEM((NP,), jnp.int32),
        pltpu.VMEM((NPAD,), jnp.int32),
        pltpu.VMEM((NPAD,), jnp.float32),
        pltpu.VMEM((32,), jnp.int32),
        pltpu.VMEM_SHARED((NPAD,), jnp.int32),
        pltpu.VMEM((64,), jnp.int32),
        pltpu.VMEM((64, H), jnp.float32),
        pltpu.SemaphoreType.DMA,
    ]

    @functools.partial(
        pl.kernel, mesh=mesh, out_type=out_type, scratch_types=scratch,
        compiler_params=pltpu.CompilerParams(needs_layout_passes=False))
    def body(i1h, i2h, g1h, g2h, xh, xs, gs, te, pos1, pos2,
             ep, gp, ranks, posb, ptok, gsort, tebuf, shp,
             myidx, rows, sem):
        c = lax.axis_index("c")
        s = lax.axis_index("s")
        io16 = lax.iota(jnp.int32, 16)

        @pl.when(s == 0)
        def _leader():
            pltpu.sync_copy(i1h, ep.at[pl.ds(0, S)])
            pltpu.sync_copy(i2h, ep.at[pl.ds(S, S)])
            pltpu.sync_copy(g1h, gp.at[pl.ds(0, S)])
            pltpu.sync_copy(g2h, gp.at[pl.ds(S, S)])
            zi = jnp.zeros((16,), jnp.int32)
            zf = jnp.zeros((16,), jnp.float32)

            def init_body(i, carry):
                ptok[pl.ds(i * 16, 16)] = zi
                gsort[pl.ds(i * 16, 16)] = zf
                return carry

            lax.fori_loop(0, NPAD // 16, init_body, 0)

            def group_a(g, carry):
                pidx, cnt = carry
                ev = ep[pl.ds(g * 16, 16)]
                cbase = _vgather16(cnt, ev)
                rank = jnp.zeros((16,), jnp.int32)
                tot = jnp.zeros((16,), jnp.int32)
                for e in range(E):
                    m = ev == e
                    mi = m.astype(jnp.int32)
                    incl = plsc.cumsum(mi)
                    rank = jnp.where(m, incl - 1, rank)
                    tote = plsc.cummax(lax.rev(incl, (0,)))
                    tot = jnp.where(io16 == e, tote, tot)
                ranks[pl.ds(g * 16, 16)] = rank + cbase
                return (pidx + 16, cnt + tot)

            _, counts_v = lax.fori_loop(
                0, NP // 16, group_a, (io16, jnp.zeros((16,), jnp.int32)))

            ntiles_v = (counts_v + (T - 1)) // T
            tstart_v = plsc.cumsum(ntiles_v) - ntiles_v
            basevals = tstart_v * T
            for j in range(2):
                tv = io16 + j * 16
                ind = jnp.zeros((16,), jnp.int32)
                for e in range(E):
                    tse = _vgather16(tstart_v, jnp.full((16,), e, jnp.int32))
                    ind = ind + jnp.where(tv >= tse, 1, 0)
                tebuf[pl.ds(j * 16, 16)] = jnp.maximum(ind - 1, 0)

            def group_b(g, pidx):
                ev = ep[pl.ds(g * 16, 16)]
                rk = ranks[pl.ds(g * 16, 16)]
                gv = gp[pl.ds(g * 16, 16)]
                base = _vgather16(basevals, ev)
                posv = base + rk
                tokv = pidx & (S - 1)
                plsc.store_scatter(ptok, [posv], tokv)
                plsc.store_scatter(gsort, [posv], gv)
                posb[pl.ds(g * 16, 16)] = posv
                return pidx + 16

            lax.fori_loop(0, NP // 16, group_b, io16)
            pltpu.sync_copy(ptok, shp)

            @pl.when(c == 0)
            def _meta():
                pltpu.sync_copy(posb.at[pl.ds(0, S)], pos1)
                pltpu.sync_copy(posb.at[pl.ds(S, S)], pos2)
                pltpu.sync_copy(gsort, gs)
                pltpu.sync_copy(tebuf, te)

        plsc.subcore_barrier()

        half = NPAD // 2
        per = half // 16
        for ch in range(per // 64):
            start = c * half + s * per + ch * 64
            pltpu.sync_copy(shp.at[pl.ds(start, 64)], myidx)
            pltpu.async_copy(xh.at[myidx], rows, sem).wait()
            pltpu.sync_copy(rows, xs.at[pl.ds(start, 64)])

    return body(i1, i2, g1, g2, x2d)


def _ffn_routed_body(te_ref, xs_ref, g_ref, w1_ref, b1_ref, w2_ref, b2_ref,
                     w3_ref, b3_ref, out_ref):
    j = pl.program_id(1)
    xt = xs_ref[...].astype(jnp.bfloat16)
    gcol = g_ref[0, 0][:, None]
    h1 = lax.dot_general(xt, w1_ref[0].astype(jnp.bfloat16),
                         (((1,), (1,)), ((), ())),
                         preferred_element_type=jnp.float32) + b1_ref[0, 0]
    h3 = lax.dot_general(xt, w3_ref[0].astype(jnp.bfloat16),
                         (((1,), (1,)), ((), ())),
                         preferred_element_type=jnp.float32) + b3_ref[0, 0]
    hh = (h1 * jax.nn.sigmoid(h1) * h3).astype(jnp.bfloat16)
    y = lax.dot_general(hh, w2_ref[0].astype(jnp.bfloat16),
                        (((1,), (1,)), ((), ())),
                        preferred_element_type=jnp.float32)

    @pl.when(j == 0)
    def _():
        out_ref[...] = (y + b2_ref[0]) * gcol

    @pl.when(j != 0)
    def _():
        out_ref[...] = out_ref[...] + y * gcol


def _ffn_routed(te, xs, gs3, rw1, rb1, rw2, rb2, rw3, rb3):
    grid_spec = pltpu.PrefetchScalarGridSpec(
        num_scalar_prefetch=1,
        grid=(NT, NFC),
        in_specs=[
            pl.BlockSpec((T, H), lambda i, j, te: (i, 0)),
            pl.BlockSpec((1, 1, T), lambda i, j, te: (i, 0, 0)),
            pl.BlockSpec((1, FFC, H), lambda i, j, te: (te[i], j, 0)),
            pl.BlockSpec((1, 1, 1, FFC), lambda i, j, te: (te[i], j, 0, 0)),
            pl.BlockSpec((1, H, FFC), lambda i, j, te: (te[i], 0, j)),
            pl.BlockSpec((1, 1, H), lambda i, j, te: (te[i], 0, 0)),
            pl.BlockSpec((1, FFC, H), lambda i, j, te: (te[i], j, 0)),
            pl.BlockSpec((1, 1, 1, FFC), lambda i, j, te: (te[i], j, 0, 0)),
        ],
        out_specs=pl.BlockSpec((T, H), lambda i, j, te: (i, 0)),
    )
    return pl.pallas_call(
        _ffn_routed_body,
        grid_spec=grid_spec,
        out_shape=jax.ShapeDtypeStruct((NPAD, H), jnp.float32),
        compiler_params=pltpu.CompilerParams(
            dimension_semantics=("arbitrary", "arbitrary"),
            vmem_limit_bytes=128 * 1024 * 1024,
        ),
    )(te, xs, gs3, rw1, rb1, rw2, rb2, rw3, rb3)


def _combine_gather(pos1, pos2, yr):
    mesh = plsc.VectorSubcoreMesh(core_axis_name="c", subcore_axis_name="s")
    out_type = [
        jax.ShapeDtypeStruct((S, H), jnp.float32),
        jax.ShapeDtypeStruct((S, H), jnp.float32),
    ]
    scratch = [
        pltpu.VMEM((64,), jnp.int32),
        pltpu.VMEM((64, H), jnp.float32),
        pltpu.SemaphoreType.DMA,
    ]

    @functools.partial(pl.kernel, mesh=mesh, out_type=out_type,
                       scratch_types=scratch)
    def body(p1h, p2h, yh, a, b, myidx, rows, sem):
        c = lax.axis_index("c")
        s = lax.axis_index("s")
        base = (c * 16 + s) * 64
        pltpu.sync_copy(p1h.at[pl.ds(base, 64)], myidx)
        pltpu.async_copy(yh.at[myidx], rows, sem).wait()
        pltpu.sync_copy(rows, a.at[pl.ds(base, 64)])
        pltpu.sync_copy(p2h.at[pl.ds(base, 64)], myidx)
        pltpu.async_copy(yh.at[myidx], rows, sem).wait()
        pltpu.sync_copy(rows, b.at[pl.ds(base, 64)])

    return body(pos1, pos2, yr)


def _ffn_shared_body(x_ref, w1_ref, b1_ref, w2_ref, b2_ref, w3_ref, b3_ref,
                     a_ref, b2r_ref, out_ref):
    j = pl.program_id(1)
    xt = x_ref[...].astype(jnp.bfloat16)
    h1 = lax.dot_general(xt, w1_ref[...].astype(jnp.bfloat16),
                         (((1,), (1,)), ((), ())),
                         preferred_element_type=jnp.float32) + b1_ref[0]
    h3 = lax.dot_general(xt, w3_ref[...].astype(jnp.bfloat16),
                         (((1,), (1,)), ((), ())),
                         preferred_element_type=jnp.float32) + b3_ref[0]
    hh = (h1 * jax.nn.sigmoid(h1) * h3).astype(jnp.bfloat16)
    y = lax.dot_general(hh, w2_ref[...].astype(jnp.bfloat16),
                        (((1,), (1,)), ((), ())),
                        preferred_element_type=jnp.float32)

    @pl.when(j == 0)
    def _():
        out_ref[...] = y + b2_ref[...][None, :] + a_ref[...] + b2r_ref[...]

    @pl.when(j != 0)
    def _():
        out_ref[...] = out_ref[...] + y


def _ffn_shared(x2d, sw1, sb1, sw2, sb2, sw3, sb3, a, b):
    return pl.pallas_call(
        _ffn_shared_body,
        grid=(ST, NFC),
        in_specs=[
            pl.BlockSpec((T, H), lambda i, j: (i, 0)),
            pl.BlockSpec((FFC, H), lambda i, j: (j, 0)),
            pl.BlockSpec((1, 1, FFC), lambda i, j: (j, 0, 0)),
            pl.BlockSpec((H, FFC), lambda i, j: (0, j)),
            pl.BlockSpec((H,), lambda i, j: (0,)),
            pl.BlockSpec((FFC, H), lambda i, j: (j, 0)),
            pl.BlockSpec((1, 1, FFC), lambda i, j: (j, 0, 0)),
            pl.BlockSpec((T, H), lambda i, j: (i, 0)),
            pl.BlockSpec((T, H), lambda i, j: (i, 0)),
        ],
        out_specs=pl.BlockSpec((T, H), lambda i, j: (i, 0)),
        out_shape=jax.ShapeDtypeStruct((S, H), jnp.float32),
        compiler_params=pltpu.CompilerParams(
            dimension_semantics=("arbitrary", "arbitrary"),
            vmem_limit_bytes=128 * 1024 * 1024,
        ),
    )(x2d, sw1, sb1, sw2, sb2, sw3, sb3, a, b)


def kernel(x, router_w, router_b, expert_biases, sw1, sb1, sw2, sb2, sw3, sb3,
           rw1, rb1, rw2, rb2, rw3, rb3):
    x2d = x.reshape(S, H)
    i1, i2, g1, g2 = _router(x2d, router_w, router_b, expert_biases)
    xs, gs, te, pos1, pos2 = _dispatch(i1, i2, g1, g2, x2d)
    gs3 = gs.reshape(NT, 1, T)
    rb1r = rb1.reshape(E, NFC, 1, FFC)
    rb3r = rb3.reshape(E, NFC, 1, FFC)
    rb2r = rb2.reshape(E, 1, H)
    yr = _ffn_routed(te, xs, gs3, rw1, rb1r, rw2, rb2r, rw3, rb3r)
    a, b = _combine_gather(pos1, pos2, yr)
    sb1r = sb1.reshape(NFC, 1, FFC)
    sb3r = sb3.reshape(NFC, 1, FFC)
    out = _ffn_shared(x2d, sw1, sb1r, sw2, sb2, sw3, sb3r, a, b)
    return out.reshape(1, S, H)
